# baseline (speedup 1.0000x reference)
import jax
import jax.numpy as jnp
from jax import lax
from jax.experimental import pallas as pl
from jax.experimental.pallas import tpu as pltpu

_NEG_POS_RATIO = 3


def _stage_a_kernel(clo_ref, chi_ref, lab_ref, pred_ref, gt_ref,
                    s_ref, ep_ref, e0_ref, sl1_ref):
    clo = clo_ref[0, 0]
    chi = chi_ref[0, 0]
    C = clo.shape[1]
    eye = (lax.broadcasted_iota(jnp.int32, (C, C), 0) ==
           lax.broadcasted_iota(jnp.int32, (C, C), 1)).astype(jnp.float32)
    dn = (((1,), (1,)), ((), ()))
    cT = jnp.concatenate(
        [lax.dot_general(eye, clo, dn, preferred_element_type=jnp.float32),
         lax.dot_general(eye, chi, dn, preferred_element_type=jnp.float32)],
        axis=1)
    P = cT.shape[1]
    eT = jnp.exp(cT)
    lab = lab_ref[0]
    s_ref[0] = jnp.sum(eT, axis=0, keepdims=True)
    sub_iota = lax.broadcasted_iota(jnp.int32, (C, P), 0)
    ep_ref[0] = jnp.sum(jnp.where(sub_iota == lab, eT, 0.0),
                        axis=0, keepdims=True)
    e0_ref[0] = eT[0:1, :]
    d = pred_ref[0] - gt_ref[0]
    ad = jnp.abs(d)
    sl1 = jnp.where(ad < 1.0, 0.5 * d * d, ad - 0.5)
    pos = lab > 0
    sl1_ref[...] = jnp.sum(jnp.where(pos, sl1, 0.0)).reshape(1, 1, 1)


def _stage_b_kernel(s_ref, ep_ref, e0_ref, lab_ref, sl1p_ref,
                    out_sl1_ref, out_cls_ref):
    lab = lab_ref[...]
    B, P = lab.shape
    pos = lab > 0
    posf = pos.astype(jnp.float32)
    npos = jnp.sum(posf, axis=1, keepdims=True)
    k = jnp.minimum(_NEG_POS_RATIO * npos, jnp.float32(P) - npos)
    s = s_ref[...]
    ce = jnp.log(s / ep_ref[...])
    r0 = s / e0_ref[...]
    u = lax.bitcast_convert_type(r0, jnp.uint32)
    u = jnp.where(pos, jnp.uint32(0), u)
    T = jnp.zeros((B, 1), jnp.uint32)
    for bit in range(31, -1, -1):
        cand = T | jnp.uint32(2 ** bit)
        cnt = jnp.sum(jnp.where(u >= cand, 1.0, 0.0), axis=1, keepdims=True)
        T = jnp.where(cnt >= k, cand, T)
    count_gt = jnp.sum(jnp.where(u > T, 1.0, 0.0), axis=1, keepdims=True)
    need = k - count_gt
    eq = u == T
    idx = lax.broadcasted_iota(jnp.int32, (B, P), 1)
    I = jnp.zeros((B, 1), jnp.int32)
    for bit in range(13, -1, -1):
        cand = I + (1 << bit)
        cnt = jnp.sum(jnp.where(eq & (idx < cand), 1.0, 0.0),
                      axis=1, keepdims=True)
        I = jnp.where(cnt <= need, cand, I)
    mask = pos | (u > T) | (eq & (idx < I))
    cls = jnp.sum(jnp.where(mask, ce, 0.0))
    npos_tot = jnp.sum(posf)
    out_sl1_ref[...] = (jnp.sum(sl1p_ref[...]) / npos_tot).reshape(1, 1)
    out_cls_ref[...] = (cls / npos_tot).reshape(1, 1)


def kernel(confidence, predicted_locations, labels, gt_locations):
    B, P, C = confidence.shape
    H = P // 2
    conf4 = confidence.reshape(B, 2, H, C)
    predT = jnp.transpose(predicted_locations, (0, 2, 1))
    gtT = jnp.transpose(gt_locations, (0, 2, 1))
    lab3 = labels.reshape(B, 1, P)

    s3, ep3, e03, sl1p = pl.pallas_call(
        _stage_a_kernel,
        grid=(B,),
        in_specs=[
            pl.BlockSpec((1, 1, H, C), lambda b: (b, 0, 0, 0)),
            pl.BlockSpec((1, 1, H, C), lambda b: (b, 1, 0, 0)),
            pl.BlockSpec((1, 1, P), lambda b: (b, 0, 0)),
            pl.BlockSpec((1, 4, P), lambda b: (b, 0, 0)),
            pl.BlockSpec((1, 4, P), lambda b: (b, 0, 0)),
        ],
        out_specs=[
            pl.BlockSpec((1, 1, P), lambda b: (b, 0, 0)),
            pl.BlockSpec((1, 1, P), lambda b: (b, 0, 0)),
            pl.BlockSpec((1, 1, P), lambda b: (b, 0, 0)),
            pl.BlockSpec((1, 1, 1), lambda b: (b, 0, 0)),
        ],
        out_shape=[
            jax.ShapeDtypeStruct((B, 1, P), jnp.float32),
            jax.ShapeDtypeStruct((B, 1, P), jnp.float32),
            jax.ShapeDtypeStruct((B, 1, P), jnp.float32),
            jax.ShapeDtypeStruct((B, 1, 1), jnp.float32),
        ],
    )(conf4, conf4, lab3, predT, gtT)

    out_sl1, out_cls = pl.pallas_call(
        _stage_b_kernel,
        out_shape=[
            jax.ShapeDtypeStruct((1, 1), jnp.float32),
            jax.ShapeDtypeStruct((1, 1), jnp.float32),
        ],
    )(s3.reshape(B, P), ep3.reshape(B, P), e03.reshape(B, P),
      labels, sl1p.reshape(B, 1))

    return (out_sl1[0, 0], out_cls[0, 0])


# speedup vs baseline: 2.6063x; 2.6063x over previous
import jax
import jax.numpy as jnp
from jax import lax
from jax.experimental import pallas as pl
from jax.experimental.pallas import tpu as pltpu

_NEG_POS_RATIO = 3


def _stage_a_kernel(conf_ref, lab_ref, pred_ref, gt_ref,
                    s_ref, ep_ref, e0_ref, sl1_ref):
    c = conf_ref[0]
    P, C = c.shape
    eye = (lax.broadcasted_iota(jnp.int32, (C, C), 0) ==
           lax.broadcasted_iota(jnp.int32, (C, C), 1)).astype(jnp.float32)
    cT = lax.dot_general(eye, c, (((1,), (1,)), ((), ())),
                         preferred_element_type=jnp.float32)
    eT = jnp.exp(cT)
    lab = lab_ref[0]
    s_ref[0] = jnp.sum(eT, axis=0, keepdims=True)
    sub_iota = lax.broadcasted_iota(jnp.int32, (C, P), 0)
    ep_ref[0] = jnp.sum(jnp.where(sub_iota == lab, eT, 0.0),
                        axis=0, keepdims=True)
    e0_ref[0] = eT[0:1, :]
    d = pred_ref[0] - gt_ref[0]
    ad = jnp.abs(d)
    sl1 = jnp.where(ad < 1.0, 0.5 * d * d, ad - 0.5)
    pos = lab > 0
    sl1_ref[...] = jnp.sum(jnp.where(pos, sl1, 0.0)).reshape(1, 1, 1)


def _stage_b_kernel(s_ref, ep_ref, e0_ref, lab_ref, sl1p_ref,
                    out_sl1_ref, out_cls_ref):
    lab = lab_ref[...]
    B, P = lab.shape
    pos = lab > 0
    posf = pos.astype(jnp.float32)
    npos = jnp.sum(posf, axis=1, keepdims=True)
    k = jnp.minimum(_NEG_POS_RATIO * npos, jnp.float32(P) - npos)
    s = s_ref[...]
    ce = jnp.log(s / ep_ref[...])
    r0 = s / e0_ref[...]
    u = lax.bitcast_convert_type(r0, jnp.uint32)
    u = jnp.where(pos, jnp.uint32(0), u)
    T = jnp.zeros((B, 1), jnp.uint32)
    for bit in range(31, -1, -1):
        cand = T | jnp.uint32(2 ** bit)
        cnt = jnp.sum(jnp.where(u >= cand, 1.0, 0.0), axis=1, keepdims=True)
        T = jnp.where(cnt >= k, cand, T)
    count_gt = jnp.sum(jnp.where(u > T, 1.0, 0.0), axis=1, keepdims=True)
    need = k - count_gt
    eq = u == T
    idx = lax.broadcasted_iota(jnp.int32, (B, P), 1)
    I = jnp.zeros((B, 1), jnp.int32)
    for bit in range(13, -1, -1):
        cand = I + (1 << bit)
        cnt = jnp.sum(jnp.where(eq & (idx < cand), 1.0, 0.0),
                      axis=1, keepdims=True)
        I = jnp.where(cnt <= need, cand, I)
    mask = pos | (u > T) | (eq & (idx < I))
    cls = jnp.sum(jnp.where(mask, ce, 0.0))
    npos_tot = jnp.sum(posf)
    out_sl1_ref[...] = (jnp.sum(sl1p_ref[...]) / npos_tot).reshape(1, 1)
    out_cls_ref[...] = (cls / npos_tot).reshape(1, 1)


def kernel(confidence, predicted_locations, labels, gt_locations):
    B, P, C = confidence.shape
    predT = jnp.transpose(predicted_locations, (0, 2, 1))
    gtT = jnp.transpose(gt_locations, (0, 2, 1))
    lab3 = labels.reshape(B, 1, P)

    s3, ep3, e03, sl1p = pl.pallas_call(
        _stage_a_kernel,
        grid=(B,),
        in_specs=[
            pl.BlockSpec((1, P, C), lambda b: (b, 0, 0)),
            pl.BlockSpec((1, 1, P), lambda b: (b, 0, 0)),
            pl.BlockSpec((1, 4, P), lambda b: (b, 0, 0)),
            pl.BlockSpec((1, 4, P), lambda b: (b, 0, 0)),
        ],
        out_specs=[
            pl.BlockSpec((1, 1, P), lambda b: (b, 0, 0)),
            pl.BlockSpec((1, 1, P), lambda b: (b, 0, 0)),
            pl.BlockSpec((1, 1, P), lambda b: (b, 0, 0)),
            pl.BlockSpec((1, 1, 1), lambda b: (b, 0, 0)),
        ],
        out_shape=[
            jax.ShapeDtypeStruct((B, 1, P), jnp.float32),
            jax.ShapeDtypeStruct((B, 1, P), jnp.float32),
            jax.ShapeDtypeStruct((B, 1, P), jnp.float32),
            jax.ShapeDtypeStruct((B, 1, 1), jnp.float32),
        ],
    )(confidence, lab3, predT, gtT)

    out_sl1, out_cls = pl.pallas_call(
        _stage_b_kernel,
        out_shape=[
            jax.ShapeDtypeStruct((1, 1), jnp.float32),
            jax.ShapeDtypeStruct((1, 1), jnp.float32),
        ],
    )(s3.reshape(B, P), ep3.reshape(B, P), e03.reshape(B, P),
      labels, sl1p.reshape(B, 1))

    return (out_sl1[0, 0], out_cls[0, 0])
